# Initial kernel scaffold; baseline (speedup 1.0000x reference)
import jax
import jax.numpy as jnp
from jax.experimental import pallas as pl


def kernel(x, edge_index, W1, b1, W2, b2):
    raise NotImplementedError("write your pallas kernel here")


# speedup vs baseline: 14.4044x; 14.4044x over previous
import functools

import jax
import jax.numpy as jnp
from jax import lax
from jax.experimental import pallas as pl
from jax.experimental.pallas import tpu as pltpu
from jax.experimental.pallas import tpu_sc as plsc

N = 10000
D = 128
NPAD = 10240
NC = 2
NS = 16
NW = NC * NS
E = 320000
EPW = 10240
EPAD = NW * EPW
CH = 128
NCHUNK = EPW // CH
RPT = NPAD // NS
BR = 256

_sc_mesh = plsc.VectorSubcoreMesh(core_axis_name="c", subcore_axis_name="s")


@functools.partial(
    pl.kernel,
    out_type=jax.ShapeDtypeStruct((NC, NPAD), jnp.float32),
    mesh=_sc_mesh,
    scratch_types=[
        pltpu.VMEM((CH,), jnp.int32),
        pltpu.VMEM((CH,), jnp.float32),
        pltpu.VMEM((NPAD,), jnp.float32),
        pltpu.VMEM_SHARED((NPAD,), jnp.float32),
    ],
)
def _deg_kernel(dst_hbm, ones_hbm, zeros_hbm, out_hbm, idx_v, ones_v, bounce_v, hist_s):
    c = lax.axis_index("c")
    s = lax.axis_index("s")
    wid = c * NS + s

    @pl.when(s == 0)
    def _():
        pltpu.sync_copy(zeros_hbm, bounce_v)
        pltpu.sync_copy(bounce_v, hist_s)

    pltpu.sync_copy(ones_hbm, ones_v)
    plsc.subcore_barrier()

    def chunk(j, carry):
        base = wid * EPW + j * CH
        pltpu.sync_copy(dst_hbm.at[pl.ds(base, CH)], idx_v)
        pltpu.sync_copy(ones_v, hist_s.at[idx_v], add=True)
        return carry

    lax.fori_loop(0, NCHUNK, chunk, 0)
    plsc.subcore_barrier()

    @pl.when(s == 0)
    def _():
        pltpu.sync_copy(hist_s, bounce_v)
        pltpu.sync_copy(bounce_v, out_hbm.at[c])


@functools.partial(
    pl.kernel,
    out_type=jax.ShapeDtypeStruct((NC, NPAD, D), jnp.float32),
    mesh=_sc_mesh,
    scratch_types=[
        pltpu.VMEM((CH,), jnp.int32),
        pltpu.VMEM((CH,), jnp.int32),
        pltpu.VMEM((CH, D), jnp.float32),
        pltpu.VMEM_SHARED((NPAD, D), jnp.float32),
        pltpu.SemaphoreType.DMA,
    ],
)
def _agg_kernel(g_hbm, src_hbm, dst_hbm, zrows_hbm, out_hbm, sidx, didx, rows, acc, sem):
    c = lax.axis_index("c")
    s = lax.axis_index("s")
    wid = c * NS + s

    pltpu.sync_copy(zrows_hbm, rows)
    for k in range(RPT // CH):
        pltpu.sync_copy(rows, acc.at[pl.ds(s * RPT + k * CH, CH)])
    plsc.subcore_barrier()

    def chunk(j, carry):
        base = wid * EPW + j * CH
        pltpu.sync_copy(src_hbm.at[pl.ds(base, CH)], sidx)
        pltpu.sync_copy(dst_hbm.at[pl.ds(base, CH)], didx)
        pltpu.async_copy(g_hbm.at[sidx], rows, sem).wait()
        pltpu.sync_copy(rows, acc.at[didx], add=True)
        return carry

    lax.fori_loop(0, NCHUNK, chunk, 0)
    plsc.subcore_barrier()

    for k in range(RPT // CH):
        r0 = s * RPT + k * CH
        pltpu.sync_copy(acc.at[pl.ds(r0, CH)], rows)
        pltpu.sync_copy(rows, out_hbm.at[c, pl.ds(r0, CH), :])


def _tc1_body(cnt_ref, x_ref, w_ref, g_ref, dis_ref):
    dis = lax.rsqrt(cnt_ref[...] + 1.0)
    u = jnp.dot(x_ref[...], w_ref[...],
                preferred_element_type=jnp.float32,
                precision=lax.Precision.HIGHEST)
    g_ref[...] = u * dis
    dis_ref[...] = dis


def _tc1(cnt_col, x_pad, W1):
    return pl.pallas_call(
        _tc1_body,
        grid=(NPAD // BR,),
        in_specs=[
            pl.BlockSpec((BR, 1), lambda i: (i, 0)),
            pl.BlockSpec((BR, D), lambda i: (i, 0)),
            pl.BlockSpec((D, D), lambda i: (0, 0)),
        ],
        out_specs=[
            pl.BlockSpec((BR, D), lambda i: (i, 0)),
            pl.BlockSpec((BR, 1), lambda i: (i, 0)),
        ],
        out_shape=[
            jax.ShapeDtypeStruct((NPAD, D), jnp.float32),
            jax.ShapeDtypeStruct((NPAD, 1), jnp.float32),
        ],
    )(cnt_col, x_pad, W1)


def _tc2_body(a_ref, b_ref, g_ref, dis_ref, bias_ref, w_ref, out_ref):
    dis = dis_ref[...]
    h = jnp.maximum(dis * (a_ref[...] + b_ref[...] + g_ref[...]) + bias_ref[...], 0.0)
    u = jnp.dot(h, w_ref[...],
                preferred_element_type=jnp.float32,
                precision=lax.Precision.HIGHEST)
    out_ref[...] = u * dis


def _tc2(acc_a, acc_b, g1, dis_col, b1, W2):
    return pl.pallas_call(
        _tc2_body,
        grid=(NPAD // BR,),
        in_specs=[
            pl.BlockSpec((BR, D), lambda i: (i, 0)),
            pl.BlockSpec((BR, D), lambda i: (i, 0)),
            pl.BlockSpec((BR, D), lambda i: (i, 0)),
            pl.BlockSpec((BR, 1), lambda i: (i, 0)),
            pl.BlockSpec((1, D), lambda i: (0, 0)),
            pl.BlockSpec((D, D), lambda i: (0, 0)),
        ],
        out_specs=pl.BlockSpec((BR, D), lambda i: (i, 0)),
        out_shape=jax.ShapeDtypeStruct((NPAD, D), jnp.float32),
    )(acc_a, acc_b, g1, dis_col, b1, W2)


def _tc3_body(a_ref, b_ref, g_ref, dis_ref, bias_ref, out_ref):
    out_ref[...] = jnp.maximum(
        dis_ref[...] * (a_ref[...] + b_ref[...] + g_ref[...]) + bias_ref[...], 0.0)


def _tc3(acc_a, acc_b, g2, dis_col, b2):
    return pl.pallas_call(
        _tc3_body,
        grid=(NPAD // BR,),
        in_specs=[
            pl.BlockSpec((BR, D), lambda i: (i, 0)),
            pl.BlockSpec((BR, D), lambda i: (i, 0)),
            pl.BlockSpec((BR, D), lambda i: (i, 0)),
            pl.BlockSpec((BR, 1), lambda i: (i, 0)),
            pl.BlockSpec((1, D), lambda i: (0, 0)),
        ],
        out_specs=pl.BlockSpec((BR, D), lambda i: (i, 0)),
        out_shape=jax.ShapeDtypeStruct((NPAD, D), jnp.float32),
    )(acc_a, acc_b, g2, dis_col, b2)


@jax.jit
def kernel(x, edge_index, W1, b1, W2, b2):
    src = edge_index[0]
    dst = edge_index[1]
    pad_idx = N + (jnp.arange(EPAD - E, dtype=jnp.int32) % (NPAD - N))
    srcp = jnp.concatenate([src, pad_idx])
    dstp = jnp.concatenate([dst, pad_idx])
    x_pad = jnp.pad(x, ((0, NPAD - N), (0, 0)))

    ones_ch = jnp.ones((CH,), jnp.float32)
    zeros_hist = jnp.zeros((NPAD,), jnp.float32)
    zeros_rows = jnp.zeros((CH, D), jnp.float32)

    cnt = _deg_kernel(dstp, ones_ch, zeros_hist)
    cnt_col = (cnt[0] + cnt[1]).reshape(NPAD, 1)

    g1, dis_col = _tc1(cnt_col, x_pad, W1)
    acc1 = _agg_kernel(g1, srcp, dstp, zeros_rows)
    g2 = _tc2(acc1[0], acc1[1], g1, dis_col, b1.reshape(1, D), W2)
    acc2 = _agg_kernel(g2, srcp, dstp, zeros_rows)
    out = _tc3(acc2[0], acc2[1], g2, dis_col, b2.reshape(1, D))
    return out[:N]


# speedup vs baseline: 26.0545x; 1.8088x over previous
import functools

import jax
import jax.numpy as jnp
from jax import lax
from jax.experimental import pallas as pl
from jax.experimental.pallas import tpu as pltpu
from jax.experimental.pallas import tpu_sc as plsc

N = 10000
D = 128
NPAD = 10240
NC = 2
NS = 16
NW = NC * NS
E = 320000
EPW = 10240
EPAD = NW * EPW
CH = 128
NCHUNK = EPW // CH
RPT = NPAD // NS
BR = 256

_sc_mesh = plsc.VectorSubcoreMesh(core_axis_name="c", subcore_axis_name="s")


@functools.partial(
    pl.kernel,
    out_type=jax.ShapeDtypeStruct((NC, NPAD), jnp.float32),
    mesh=_sc_mesh,
    scratch_types=[
        pltpu.VMEM((NCHUNK, CH), jnp.int32),
        pltpu.VMEM((CH,), jnp.float32),
        pltpu.VMEM((NPAD,), jnp.float32),
        pltpu.VMEM_SHARED((NPAD,), jnp.float32),
    ],
)
def _deg_kernel(dst_hbm, ones_hbm, zeros_hbm, out_hbm, idx_v, ones_v, bounce_v, hist_s):
    c = lax.axis_index("c")
    s = lax.axis_index("s")
    wid = c * NS + s

    @pl.when(s == 0)
    def _():
        pltpu.sync_copy(zeros_hbm, bounce_v)
        pltpu.sync_copy(bounce_v, hist_s)

    pltpu.sync_copy(dst_hbm.at[wid], idx_v)
    pltpu.sync_copy(ones_hbm, ones_v)
    plsc.subcore_barrier()

    def chunk(j, carry):
        pltpu.sync_copy(ones_v, hist_s.at[idx_v.at[j]], add=True)
        return carry

    lax.fori_loop(0, NCHUNK, chunk, 0)
    plsc.subcore_barrier()

    @pl.when(s == 0)
    def _():
        pltpu.sync_copy(hist_s, bounce_v)
        pltpu.sync_copy(bounce_v, out_hbm.at[c])


NBUF = 2


@functools.partial(
    pl.kernel,
    out_type=jax.ShapeDtypeStruct((NC, NPAD, D), jnp.float32),
    mesh=_sc_mesh,
    scratch_types=[
        pltpu.VMEM((NCHUNK, CH), jnp.int32),
        pltpu.VMEM((CH,), jnp.int32),
        pltpu.VMEM((CH,), jnp.int32),
        pltpu.VMEM((CH,), jnp.int32),
        pltpu.VMEM((CH,), jnp.int32),
        pltpu.VMEM((CH, D), jnp.float32),
        pltpu.VMEM((CH, D), jnp.float32),
        pltpu.VMEM_SHARED((NPAD, D), jnp.float32),
        pltpu.SemaphoreType.DMA,
        pltpu.SemaphoreType.DMA,
    ],
)
def _agg_kernel(g_hbm, packed_hbm, zrows_hbm, out_hbm, pidx,
                s0, s1, d0, d1, r0, r1, acc, m0, m1):
    sidx = (s0, s1)
    didx = (d0, d1)
    rows = (r0, r1)
    sems = (m0, m1)
    c = lax.axis_index("c")
    s = lax.axis_index("s")
    wid = c * NS + s

    def unpack(j, k):
        for l in range(CH // 16):
            v = pidx[j, pl.ds(l * 16, 16)]
            sidx[k][pl.ds(l * 16, 16)] = v & 0x3FFF
            didx[k][pl.ds(l * 16, 16)] = lax.shift_right_logical(v, 14)

    pltpu.sync_copy(packed_hbm.at[wid], pidx)
    pltpu.sync_copy(zrows_hbm, rows[0])
    for k in range(RPT // CH):
        pltpu.sync_copy(rows[0], acc.at[pl.ds(s * RPT + k * CH, CH)])
    plsc.subcore_barrier()

    for k in range(NBUF):
        unpack(k, k)
        pltpu.async_copy(g_hbm.at[sidx[k]], rows[k], sems[k])

    def pair(q, carry):
        j0 = q * NBUF
        for k in range(NBUF):
            j = j0 + k
            pltpu.make_async_copy(g_hbm.at[sidx[k]], rows[k], sems[k]).wait()
            pltpu.sync_copy(rows[k], acc.at[didx[k]], add=True)

            @pl.when(j + NBUF < NCHUNK)
            def _():
                unpack(j + NBUF, k)
                pltpu.async_copy(g_hbm.at[sidx[k]], rows[k], sems[k])

        return carry

    lax.fori_loop(0, NCHUNK // NBUF, pair, 0)
    plsc.subcore_barrier()

    for k in range(RPT // CH):
        rb = s * RPT + k * CH
        pltpu.sync_copy(acc.at[pl.ds(rb, CH)], rows[0])
        pltpu.sync_copy(rows[0], out_hbm.at[c, pl.ds(rb, CH), :])


def _tc1_body(cnt_ref, x_ref, w_ref, g_ref, dis_ref):
    dis = lax.rsqrt(cnt_ref[...] + 1.0)
    u = jnp.dot(x_ref[...], w_ref[...],
                preferred_element_type=jnp.float32,
                precision=lax.Precision.HIGHEST)
    g_ref[...] = u * dis
    dis_ref[...] = dis


def _tc1(cnt_col, x_pad, W1):
    return pl.pallas_call(
        _tc1_body,
        grid=(NPAD // BR,),
        in_specs=[
            pl.BlockSpec((BR, 1), lambda i: (i, 0)),
            pl.BlockSpec((BR, D), lambda i: (i, 0)),
            pl.BlockSpec((D, D), lambda i: (0, 0)),
        ],
        out_specs=[
            pl.BlockSpec((BR, D), lambda i: (i, 0)),
            pl.BlockSpec((BR, 1), lambda i: (i, 0)),
        ],
        out_shape=[
            jax.ShapeDtypeStruct((NPAD, D), jnp.float32),
            jax.ShapeDtypeStruct((NPAD, 1), jnp.float32),
        ],
    )(cnt_col, x_pad, W1)


def _tc2_body(a_ref, b_ref, g_ref, dis_ref, bias_ref, w_ref, out_ref):
    dis = dis_ref[...]
    h = jnp.maximum(dis * (a_ref[...] + b_ref[...] + g_ref[...]) + bias_ref[...], 0.0)
    u = jnp.dot(h, w_ref[...],
                preferred_element_type=jnp.float32,
                precision=lax.Precision.HIGHEST)
    out_ref[...] = u * dis


def _tc2(acc_a, acc_b, g1, dis_col, b1, W2):
    return pl.pallas_call(
        _tc2_body,
        grid=(NPAD // BR,),
        in_specs=[
            pl.BlockSpec((BR, D), lambda i: (i, 0)),
            pl.BlockSpec((BR, D), lambda i: (i, 0)),
            pl.BlockSpec((BR, D), lambda i: (i, 0)),
            pl.BlockSpec((BR, 1), lambda i: (i, 0)),
            pl.BlockSpec((1, D), lambda i: (0, 0)),
            pl.BlockSpec((D, D), lambda i: (0, 0)),
        ],
        out_specs=pl.BlockSpec((BR, D), lambda i: (i, 0)),
        out_shape=jax.ShapeDtypeStruct((NPAD, D), jnp.float32),
    )(acc_a, acc_b, g1, dis_col, b1, W2)


def _tc3_body(a_ref, b_ref, g_ref, dis_ref, bias_ref, out_ref):
    out_ref[...] = jnp.maximum(
        dis_ref[...] * (a_ref[...] + b_ref[...] + g_ref[...]) + bias_ref[...], 0.0)


def _tc3(acc_a, acc_b, g2, dis_col, b2):
    return pl.pallas_call(
        _tc3_body,
        grid=(NPAD // BR,),
        in_specs=[
            pl.BlockSpec((BR, D), lambda i: (i, 0)),
            pl.BlockSpec((BR, D), lambda i: (i, 0)),
            pl.BlockSpec((BR, D), lambda i: (i, 0)),
            pl.BlockSpec((BR, 1), lambda i: (i, 0)),
            pl.BlockSpec((1, D), lambda i: (0, 0)),
        ],
        out_specs=pl.BlockSpec((BR, D), lambda i: (i, 0)),
        out_shape=jax.ShapeDtypeStruct((NPAD, D), jnp.float32),
    )(acc_a, acc_b, g2, dis_col, b2)


@jax.jit
def kernel(x, edge_index, W1, b1, W2, b2):
    src = edge_index[0]
    dst = edge_index[1]
    pad_idx = N + (jnp.arange(EPAD - E, dtype=jnp.int32) % (NPAD - N))
    srcp = jnp.concatenate([src, pad_idx])
    dstp = jnp.concatenate([dst, pad_idx])
    packed = (srcp | (dstp << 14)).reshape(NW, NCHUNK, CH)
    dst3d = dstp.reshape(NW, NCHUNK, CH)
    x_pad = jnp.pad(x, ((0, NPAD - N), (0, 0)))

    ones_ch = jnp.ones((CH,), jnp.float32)
    zeros_hist = jnp.zeros((NPAD,), jnp.float32)
    zeros_rows = jnp.zeros((CH, D), jnp.float32)

    cnt = _deg_kernel(dst3d, ones_ch, zeros_hist)
    cnt_col = (cnt[0] + cnt[1]).reshape(NPAD, 1)

    g1, dis_col = _tc1(cnt_col, x_pad, W1)
    acc1 = _agg_kernel(g1, packed, zeros_rows)
    g2 = _tc2(acc1[0], acc1[1], g1, dis_col, b1.reshape(1, D), W2)
    acc2 = _agg_kernel(g2, packed, zeros_rows)
    out = _tc3(acc2[0], acc2[1], g2, dis_col, b2.reshape(1, D))
    return out[:N]


# speedup vs baseline: 32.2058x; 1.2361x over previous
import functools

import jax
import jax.numpy as jnp
import numpy as np
from jax import lax
from jax.experimental import pallas as pl
from jax.experimental.pallas import tpu as pltpu
from jax.experimental.pallas import tpu_sc as plsc

N = 10000
D = 128
NPAD = 10240
NC = 2
NS = 16
NW = NC * NS
E = 320000
EPW = 10240
EPAD = NW * EPW
CH = 128
NCHUNK = EPW // CH
ZPT = NPAD // NS
WPT = 632
BR = 1000
NBUF = 2

_PAD_PACKED = np.asarray(
    (np.arange(EPAD - E) % 240) | ((N + np.arange(EPAD - E) % 240) << 14),
    dtype=np.int32)

_sc_mesh = plsc.VectorSubcoreMesh(core_axis_name="c", subcore_axis_name="s")


@functools.partial(
    pl.kernel,
    out_type=jax.ShapeDtypeStruct((NC, NPAD), jnp.float32),
    mesh=_sc_mesh,
    scratch_types=[
        pltpu.VMEM((NCHUNK, CH), jnp.int32),
        pltpu.VMEM((CH,), jnp.int32),
        pltpu.VMEM((CH,), jnp.float32),
        pltpu.VMEM((NPAD,), jnp.float32),
        pltpu.VMEM_SHARED((NPAD,), jnp.float32),
    ],
)
def _deg_kernel(packed_hbm, ones_hbm, zeros_hbm, out_hbm, pidx, didx, ones_v,
                bounce_v, hist_s):
    c = lax.axis_index("c")
    s = lax.axis_index("s")
    wid = c * NS + s

    @pl.when(s == 0)
    def _():
        pltpu.sync_copy(zeros_hbm, bounce_v)
        pltpu.sync_copy(bounce_v, hist_s)

    pltpu.sync_copy(packed_hbm.at[wid], pidx)
    pltpu.sync_copy(ones_hbm, ones_v)
    plsc.subcore_barrier()

    def chunk(j, carry):
        for l in range(CH // 16):
            didx[pl.ds(l * 16, 16)] = lax.shift_right_logical(
                pidx[j, pl.ds(l * 16, 16)], 14)
        pltpu.sync_copy(ones_v, hist_s.at[didx], add=True)
        return carry

    lax.fori_loop(0, NCHUNK, chunk, 0)
    plsc.subcore_barrier()

    @pl.when(s == 0)
    def _():
        pltpu.sync_copy(hist_s, bounce_v)
        pltpu.sync_copy(bounce_v, out_hbm.at[c])


@functools.partial(
    pl.kernel,
    out_type=jax.ShapeDtypeStruct((NC, N, D), jnp.float32),
    mesh=_sc_mesh,
    scratch_types=[
        pltpu.VMEM((NCHUNK, CH), jnp.int32),
        pltpu.VMEM((CH,), jnp.int32),
        pltpu.VMEM((CH,), jnp.int32),
        pltpu.VMEM((CH,), jnp.int32),
        pltpu.VMEM((CH,), jnp.int32),
        pltpu.VMEM((CH, D), jnp.float32),
        pltpu.VMEM((CH, D), jnp.float32),
        pltpu.VMEM_SHARED((NPAD, D), jnp.float32),
        pltpu.SemaphoreType.DMA,
        pltpu.SemaphoreType.DMA,
    ],
)
def _agg_kernel(g_hbm, packed_hbm, zrows_hbm, out_hbm, pidx,
                s0, s1, d0, d1, r0, r1, acc, m0, m1):
    sidx = (s0, s1)
    didx = (d0, d1)
    rows = (r0, r1)
    sems = (m0, m1)
    c = lax.axis_index("c")
    s = lax.axis_index("s")
    wid = c * NS + s

    def unpack(j, k):
        for l in range(CH // 16):
            v = pidx[j, pl.ds(l * 16, 16)]
            sidx[k][pl.ds(l * 16, 16)] = v & 0x3FFF
            didx[k][pl.ds(l * 16, 16)] = lax.shift_right_logical(v, 14)

    pltpu.sync_copy(packed_hbm.at[wid], pidx)
    pltpu.sync_copy(zrows_hbm, rows[0])
    for k in range(ZPT // CH):
        pltpu.sync_copy(rows[0], acc.at[pl.ds(s * ZPT + k * CH, CH)])
    plsc.subcore_barrier()

    for k in range(NBUF):
        unpack(k, k)
        pltpu.async_copy(g_hbm.at[sidx[k]], rows[k], sems[k])

    def pair(q, carry):
        j0 = q * NBUF
        for k in range(NBUF):
            j = j0 + k
            pltpu.make_async_copy(g_hbm.at[sidx[k]], rows[k], sems[k]).wait()
            pltpu.sync_copy(rows[k], acc.at[didx[k]], add=True)

            @pl.when(j + NBUF < NCHUNK)
            def _():
                unpack(j + NBUF, k)
                pltpu.async_copy(g_hbm.at[sidx[k]], rows[k], sems[k])

        return carry

    lax.fori_loop(0, NCHUNK // NBUF, pair, 0)
    plsc.subcore_barrier()

    rows_s = jnp.where(s == NS - 1, N - (NS - 1) * WPT, WPT)
    base = s * WPT
    for k in range(5):
        rb = pl.multiple_of(base + jnp.minimum(k * CH, rows_s - CH), 8)
        pltpu.sync_copy(acc.at[pl.ds(rb, CH)], rows[k % NBUF])
        pltpu.sync_copy(rows[k % NBUF], out_hbm.at[c, pl.ds(rb, CH), :])


def _mm_body(x_ref, w_ref, out_ref):
    out_ref[...] = jnp.dot(x_ref[...], w_ref[...],
                           preferred_element_type=jnp.float32)


def _tc_mm(x, W):
    return pl.pallas_call(
        _mm_body,
        grid=(N // BR,),
        in_specs=[
            pl.BlockSpec((BR, D), lambda i: (i, 0)),
            pl.BlockSpec((D, D), lambda i: (0, 0)),
        ],
        out_specs=pl.BlockSpec((BR, D), lambda i: (i, 0)),
        out_shape=jax.ShapeDtypeStruct((N, D), jnp.float32),
    )(x, W)


def _scale_body(cnt_ref, u_ref, g_ref, dis_ref):
    dis = lax.rsqrt(cnt_ref[0] + cnt_ref[1] + 1.0)
    g_ref[...] = u_ref[...] * dis
    dis_ref[...] = dis


def _tc_scale(cnt3, u1):
    return pl.pallas_call(
        _scale_body,
        grid=(N // BR,),
        in_specs=[
            pl.BlockSpec((NC, BR, 1), lambda i: (0, i, 0)),
            pl.BlockSpec((BR, D), lambda i: (i, 0)),
        ],
        out_specs=[
            pl.BlockSpec((BR, D), lambda i: (i, 0)),
            pl.BlockSpec((BR, 1), lambda i: (i, 0)),
        ],
        out_shape=[
            jax.ShapeDtypeStruct((N, D), jnp.float32),
            jax.ShapeDtypeStruct((N, 1), jnp.float32),
        ],
    )(cnt3, u1)


def _tc2_body(a_ref, g_ref, dis_ref, bias_ref, w_ref, out_ref):
    dis = dis_ref[...]
    h = jnp.maximum(dis * (a_ref[0] + a_ref[1] + g_ref[...]) + bias_ref[...], 0.0)
    out_ref[...] = jnp.dot(h, w_ref[...],
                           preferred_element_type=jnp.float32) * dis


def _tc2(acc, g1, dis_col, b1, W2):
    return pl.pallas_call(
        _tc2_body,
        grid=(N // BR,),
        in_specs=[
            pl.BlockSpec((NC, BR, D), lambda i: (0, i, 0)),
            pl.BlockSpec((BR, D), lambda i: (i, 0)),
            pl.BlockSpec((BR, 1), lambda i: (i, 0)),
            pl.BlockSpec((1, D), lambda i: (0, 0)),
            pl.BlockSpec((D, D), lambda i: (0, 0)),
        ],
        out_specs=pl.BlockSpec((BR, D), lambda i: (i, 0)),
        out_shape=jax.ShapeDtypeStruct((N, D), jnp.float32),
    )(acc, g1, dis_col, b1, W2)


def _tc3_body(a_ref, g_ref, dis_ref, bias_ref, out_ref):
    out_ref[...] = jnp.maximum(
        dis_ref[...] * (a_ref[0] + a_ref[1] + g_ref[...]) + bias_ref[...], 0.0)


def _tc3(acc, g2, dis_col, b2):
    return pl.pallas_call(
        _tc3_body,
        grid=(N // BR,),
        in_specs=[
            pl.BlockSpec((NC, BR, D), lambda i: (0, i, 0)),
            pl.BlockSpec((BR, D), lambda i: (i, 0)),
            pl.BlockSpec((BR, 1), lambda i: (i, 0)),
            pl.BlockSpec((1, D), lambda i: (0, 0)),
        ],
        out_specs=pl.BlockSpec((BR, D), lambda i: (i, 0)),
        out_shape=jax.ShapeDtypeStruct((N, D), jnp.float32),
    )(acc, g2, dis_col, b2)


@jax.jit
def kernel(x, edge_index, W1, b1, W2, b2):
    packed = jnp.concatenate(
        [edge_index[0] | (edge_index[1] << 14), jnp.asarray(_PAD_PACKED)]
    ).reshape(NW, NCHUNK, CH)

    ones_ch = jnp.ones((CH,), jnp.float32)
    zeros_hist = jnp.zeros((NPAD,), jnp.float32)
    zeros_rows = jnp.zeros((CH, D), jnp.float32)

    u1 = _tc_mm(x, W1)
    cnt = _deg_kernel(packed, ones_ch, zeros_hist)
    g1, dis_col = _tc_scale(cnt[:, :N].reshape(NC, N, 1), u1)
    acc1 = _agg_kernel(g1, packed, zeros_rows)
    g2 = _tc2(acc1, g1, dis_col, b1.reshape(1, D), W2)
    acc2 = _agg_kernel(g2, packed, zeros_rows)
    return _tc3(acc2, g2, dis_col, b2.reshape(1, D))
